# baseline (speedup 1.0000x reference)
import functools

import jax
import jax.numpy as jnp
from jax import lax
from jax.experimental import pallas as pl
from jax.experimental.pallas import tpu as pltpu
from jax.experimental.pallas import tpu_sc as plsc

TASK_SIZE = 1_000_000
EMBED_DIM = 32
BATCH = 16384

_NUM_CORES = 2
_NUM_SUBCORES = 16
_NW = _NUM_CORES * _NUM_SUBCORES
_BPW = BATCH // _NW
_CHUNK = 128
_NCHUNK = _BPW // _CHUNK


@jax.jit
def _sc_gather(x2d, table):
    mesh = plsc.VectorSubcoreMesh(core_axis_name="c", subcore_axis_name="s")

    @functools.partial(
        pl.kernel,
        mesh=mesh,
        out_type=jax.ShapeDtypeStruct((BATCH, EMBED_DIM), jnp.float32),
        scratch_types=[
            pltpu.VMEM((_NCHUNK, _CHUNK), jnp.int32),
            pltpu.VMEM((_BPW, EMBED_DIM), jnp.float32),
            pltpu.SemaphoreType.DMA,
        ],
        compiler_params=pltpu.CompilerParams(use_tc_tiling_on_sc=False),
    )
    def k(x_hbm, tbl_hbm, out_hbm, idx_v, rows_v, sem):
        wid = lax.axis_index("s") * _NUM_CORES + lax.axis_index("c")
        base = wid * _BPW
        pltpu.sync_copy(x_hbm.at[wid], idx_v)
        copies = [
            pltpu.async_copy(
                tbl_hbm.at[idx_v.at[j]],
                rows_v.at[pl.ds(j * _CHUNK, _CHUNK)],
                sem,
            )
            for j in range(_NCHUNK)
        ]
        for c in copies:
            c.wait()
        pltpu.sync_copy(rows_v, out_hbm.at[pl.ds(base, _BPW)])

    return k(x2d, table)


def kernel(x, table):
    t4 = lax.optimization_barrier(table.reshape(TASK_SIZE // 4, 4 * EMBED_DIM))
    t_rm = t4.reshape(TASK_SIZE, EMBED_DIM)
    x2d = x.astype(jnp.int32).reshape(_NW, _NCHUNK, _CHUNK)
    return _sc_gather(x2d, t_rm)


# speedup vs baseline: 1.5959x; 1.5959x over previous
import functools

import jax
import jax.numpy as jnp
from jax import lax
from jax.experimental import pallas as pl
from jax.experimental.pallas import tpu as pltpu
from jax.experimental.pallas import tpu_sc as plsc

TASK_SIZE = 1_000_000
EMBED_DIM = 32
BATCH = 16384

_NUM_CORES = 2
_NUM_SUBCORES = 16
_NW = _NUM_CORES * _NUM_SUBCORES
_BPW = BATCH // _NW
_IB = _BPW // 128
_Q = 250880
_BLK = 1024
_NBLK = _Q // _BLK
_LAST_BLK = (TASK_SIZE - 1) // _BLK


@jax.jit
def _stage_rm(tt):
    def body(i0, i1, i2, i3, o_ref):
        for s, r in enumerate((i0, i1, i2, i3)):
            o_ref[:, EMBED_DIM * s : EMBED_DIM * (s + 1)] = r[...].T

    return pl.pallas_call(
        body,
        grid=(_NBLK,),
        in_specs=[
            pl.BlockSpec(
                (EMBED_DIM, _BLK),
                functools.partial(
                    lambda s, i: (0, jnp.minimum(i + s * _NBLK, _LAST_BLK)), s
                ),
            )
            for s in range(4)
        ],
        out_specs=pl.BlockSpec((_BLK, 4 * EMBED_DIM), lambda i: (i, 0)),
        out_shape=jax.ShapeDtypeStruct((_Q, 4 * EMBED_DIM), jnp.float32),
    )(tt, tt, tt, tt)


@jax.jit
def _sc_gather(idx3, t32):
    mesh = plsc.VectorSubcoreMesh(core_axis_name="c", subcore_axis_name="s")

    @functools.partial(
        pl.kernel,
        mesh=mesh,
        out_type=jax.ShapeDtypeStruct((BATCH, EMBED_DIM), jnp.float32),
        scratch_types=[
            pltpu.VMEM((_IB, 128), jnp.int32),
            pltpu.VMEM((_BPW, EMBED_DIM), jnp.float32),
            pltpu.SemaphoreType.DMA,
        ],
        compiler_params=pltpu.CompilerParams(use_tc_tiling_on_sc=False),
    )
    def k(idx_hbm, t32_hbm, out_hbm, idx_v, rows_v, sem):
        wid = lax.axis_index("s") * _NUM_CORES + lax.axis_index("c")
        pltpu.sync_copy(idx_hbm.at[wid], idx_v)
        copies = [
            pltpu.async_copy(
                t32_hbm.at[idx_v.at[ib]],
                rows_v.at[pl.ds(ib * 128, 128)],
                sem,
            )
            for ib in range(_IB)
        ]
        for c in copies:
            c.wait()
        obase = pl.multiple_of(wid * _BPW, _BPW)
        pltpu.sync_copy(rows_v, out_hbm.at[pl.ds(obase, _BPW)])

    return k(idx3, t32)


def kernel(x, table):
    t4 = _stage_rm(table.T)
    t32 = t4.reshape(4 * _Q, EMBED_DIM)
    xi = x.astype(jnp.int32)
    idx3 = ((xi % _Q) * 4 + xi // _Q).reshape(_NW, _IB, 128)
    return _sc_gather(idx3, t32)


# speedup vs baseline: 3.9393x; 2.4684x over previous
import functools

import jax
import jax.numpy as jnp
from jax import lax
from jax.experimental import pallas as pl
from jax.experimental.pallas import tpu as pltpu
from jax.experimental.pallas import tpu_sc as plsc

TASK_SIZE = 1_000_000
EMBED_DIM = 32
BATCH = 16384

_NUM_CORES = 2
_NUM_SUBCORES = 16
_NW = _NUM_CORES * _NUM_SUBCORES
_BPW = BATCH // _NW
_IB = _BPW // 128
_Q = 253952
_BLK = 4096
_NBLK = _Q // _BLK
_LAST_BLK = (TASK_SIZE - 1) // _BLK


@jax.jit
def _stage_rm(tt):
    def body(i0, i1, i2, i3, o_ref):
        stacked = jnp.concatenate((i0[...], i1[...], i2[...], i3[...]), axis=0)
        o_ref[...] = stacked.T


    return pl.pallas_call(
        body,
        grid=(_NBLK,),
        in_specs=[
            pl.BlockSpec(
                (EMBED_DIM, _BLK),
                functools.partial(
                    lambda s, i: (0, jnp.minimum(i + s * _NBLK, _LAST_BLK)), s
                ),
            )
            for s in range(4)
        ],
        out_specs=pl.BlockSpec((_BLK, 4 * EMBED_DIM), lambda i: (i, 0)),
        out_shape=jax.ShapeDtypeStruct((_Q, 4 * EMBED_DIM), jnp.float32),
    )(tt, tt, tt, tt)


@jax.jit
def _sc_gather(idx3, t32):
    mesh = plsc.VectorSubcoreMesh(core_axis_name="c", subcore_axis_name="s")

    @functools.partial(
        pl.kernel,
        mesh=mesh,
        out_type=jax.ShapeDtypeStruct((BATCH, EMBED_DIM), jnp.float32),
        scratch_types=[
            pltpu.VMEM((_IB, 128), jnp.int32),
            pltpu.VMEM((_BPW, EMBED_DIM), jnp.float32),
            pltpu.SemaphoreType.DMA,
        ],
        compiler_params=pltpu.CompilerParams(use_tc_tiling_on_sc=False),
    )
    def k(idx_hbm, t32_hbm, out_hbm, idx_v, rows_v, sem):
        wid = lax.axis_index("s") * _NUM_CORES + lax.axis_index("c")
        pltpu.sync_copy(idx_hbm.at[wid], idx_v)
        copies = [
            pltpu.async_copy(
                t32_hbm.at[idx_v.at[ib]],
                rows_v.at[pl.ds(ib * 128, 128)],
                sem,
            )
            for ib in range(_IB)
        ]
        for c in copies:
            c.wait()
        obase = pl.multiple_of(wid * _BPW, _BPW)
        pltpu.sync_copy(rows_v, out_hbm.at[pl.ds(obase, _BPW)])

    return k(idx3, t32)


def kernel(x, table):
    t4 = _stage_rm(table.T)
    t32 = t4.reshape(4 * _Q, EMBED_DIM)
    xi = x.astype(jnp.int32)
    idx3 = ((xi % _Q) * 4 + xi // _Q).reshape(_NW, _IB, 128)
    return _sc_gather(idx3, t32)


# speedup vs baseline: 4.4040x; 1.1180x over previous
import functools

import jax
import jax.numpy as jnp
from jax import lax
from jax.experimental import pallas as pl
from jax.experimental.pallas import tpu as pltpu
from jax.experimental.pallas import tpu_sc as plsc

TASK_SIZE = 1_000_000
EMBED_DIM = 32
BATCH = 16384

_NUM_CORES = 2
_NUM_SUBCORES = 16
_NW = _NUM_CORES * _NUM_SUBCORES
_BPW = BATCH // _NW
_IB = _BPW // 128
_Q = 253952
_BLK = 8192
_NBLK = _Q // _BLK
_LAST_BLK = (TASK_SIZE - 1) // _BLK


@jax.jit
def _stage_rm(tt):
    def body(i0, i1, i2, i3, o_ref):
        stacked = jnp.concatenate((i0[...], i1[...], i2[...], i3[...]), axis=0)
        o_ref[...] = stacked.T


    return pl.pallas_call(
        body,
        grid=(_NBLK,),
        in_specs=[
            pl.BlockSpec(
                (EMBED_DIM, _BLK),
                functools.partial(
                    lambda s, i: (0, jnp.minimum(i + s * _NBLK, _LAST_BLK)), s
                ),
            )
            for s in range(4)
        ],
        out_specs=pl.BlockSpec((_BLK, 4 * EMBED_DIM), lambda i: (i, 0)),
        out_shape=jax.ShapeDtypeStruct((_Q, 4 * EMBED_DIM), jnp.float32),
    )(tt, tt, tt, tt)


@jax.jit
def _sc_gather(idx3, t32):
    mesh = plsc.VectorSubcoreMesh(core_axis_name="c", subcore_axis_name="s")

    @functools.partial(
        pl.kernel,
        mesh=mesh,
        out_type=jax.ShapeDtypeStruct((BATCH, EMBED_DIM), jnp.float32),
        scratch_types=[
            pltpu.VMEM((_IB, 128), jnp.int32),
            pltpu.VMEM((_BPW, EMBED_DIM), jnp.float32),
            pltpu.SemaphoreType.DMA,
        ],
        compiler_params=pltpu.CompilerParams(use_tc_tiling_on_sc=False),
    )
    def k(idx_hbm, t32_hbm, out_hbm, idx_v, rows_v, sem):
        wid = lax.axis_index("s") * _NUM_CORES + lax.axis_index("c")
        pltpu.sync_copy(idx_hbm.at[wid], idx_v)
        copies = [
            pltpu.async_copy(
                t32_hbm.at[idx_v.at[ib]],
                rows_v.at[pl.ds(ib * 128, 128)],
                sem,
            )
            for ib in range(_IB)
        ]
        for c in copies:
            c.wait()
        obase = pl.multiple_of(wid * _BPW, _BPW)
        pltpu.sync_copy(rows_v, out_hbm.at[pl.ds(obase, _BPW)])

    return k(idx3, t32)


def kernel(x, table):
    t4 = _stage_rm(table.T)
    t32 = t4.reshape(4 * _Q, EMBED_DIM)
    xi = x.astype(jnp.int32)
    idx3 = ((xi % _Q) * 4 + xi // _Q).reshape(_NW, _IB, 128)
    return _sc_gather(idx3, t32)


# speedup vs baseline: 4.4316x; 1.0063x over previous
import functools

import jax
import jax.numpy as jnp
from jax import lax
from jax.experimental import pallas as pl
from jax.experimental.pallas import tpu as pltpu
from jax.experimental.pallas import tpu_sc as plsc

TASK_SIZE = 1_000_000
EMBED_DIM = 32
BATCH = 16384

_NUM_CORES = 2
_NUM_SUBCORES = 16
_NW = _NUM_CORES * _NUM_SUBCORES
_BPW = BATCH // _NW
_IB = _BPW // 128
_Q = 262144
_BLK = 16384
_NBLK = _Q // _BLK
_LAST_BLK = (TASK_SIZE - 1) // _BLK


@jax.jit
def _stage_rm(tt):
    def body(i0, i1, i2, i3, o_ref):
        stacked = jnp.concatenate((i0[...], i1[...], i2[...], i3[...]), axis=0)
        o_ref[...] = stacked.T


    return pl.pallas_call(
        body,
        grid=(_NBLK,),
        in_specs=[
            pl.BlockSpec(
                (EMBED_DIM, _BLK),
                functools.partial(
                    lambda s, i: (0, jnp.minimum(i + s * _NBLK, _LAST_BLK)), s
                ),
            )
            for s in range(4)
        ],
        out_specs=pl.BlockSpec((_BLK, 4 * EMBED_DIM), lambda i: (i, 0)),
        out_shape=jax.ShapeDtypeStruct((_Q, 4 * EMBED_DIM), jnp.float32),
    )(tt, tt, tt, tt)


@jax.jit
def _sc_gather(idx3, t32):
    mesh = plsc.VectorSubcoreMesh(core_axis_name="c", subcore_axis_name="s")

    @functools.partial(
        pl.kernel,
        mesh=mesh,
        out_type=jax.ShapeDtypeStruct((BATCH, EMBED_DIM), jnp.float32),
        scratch_types=[
            pltpu.VMEM((_IB, 128), jnp.int32),
            pltpu.VMEM((_BPW, EMBED_DIM), jnp.float32),
            pltpu.SemaphoreType.DMA,
        ],
        compiler_params=pltpu.CompilerParams(use_tc_tiling_on_sc=False),
    )
    def k(idx_hbm, t32_hbm, out_hbm, idx_v, rows_v, sem):
        wid = lax.axis_index("s") * _NUM_CORES + lax.axis_index("c")
        pltpu.sync_copy(idx_hbm.at[wid], idx_v)
        copies = [
            pltpu.async_copy(
                t32_hbm.at[idx_v.at[ib]],
                rows_v.at[pl.ds(ib * 128, 128)],
                sem,
            )
            for ib in range(_IB)
        ]
        for c in copies:
            c.wait()
        obase = pl.multiple_of(wid * _BPW, _BPW)
        pltpu.sync_copy(rows_v, out_hbm.at[pl.ds(obase, _BPW)])

    return k(idx3, t32)


def kernel(x, table):
    t4 = _stage_rm(table.T)
    t32 = t4.reshape(4 * _Q, EMBED_DIM)
    xi = x.astype(jnp.int32)
    idx3 = ((xi % _Q) * 4 + xi // _Q).reshape(_NW, _IB, 128)
    return _sc_gather(idx3, t32)


# speedup vs baseline: 4.4833x; 1.0117x over previous
import functools

import jax
import jax.numpy as jnp
from jax import lax
from jax.experimental import pallas as pl
from jax.experimental.pallas import tpu as pltpu
from jax.experimental.pallas import tpu_sc as plsc

TASK_SIZE = 1_000_000
EMBED_DIM = 32
BATCH = 16384

_NUM_CORES = 2
_NUM_SUBCORES = 16
_NW = _NUM_CORES * _NUM_SUBCORES
_BPW = BATCH // _NW
_IB = _BPW // 128
_Q = 262144
_BLK = 16384
_NBLK = _Q // _BLK
_LAST_BLK = (TASK_SIZE - 1) // _BLK


@jax.jit
def _stage_rm(tt):
    def body(i0, i1, i2, i3, o_ref):
        stacked = jnp.concatenate((i0[...], i1[...], i2[...], i3[...]), axis=0)
        o_ref[...] = stacked.T


    return pl.pallas_call(
        body,
        grid=(_NBLK,),
        in_specs=[
            pl.BlockSpec(
                (EMBED_DIM, _BLK),
                functools.partial(
                    lambda s, i: (0, jnp.minimum(i + s * _NBLK, _LAST_BLK)), s
                ),
            )
            for s in range(4)
        ],
        out_specs=pl.BlockSpec((_BLK, 4 * EMBED_DIM), lambda i: (i, 0)),
        out_shape=jax.ShapeDtypeStruct((_Q, 4 * EMBED_DIM), jnp.float32),
    )(tt, tt, tt, tt)


@jax.jit
def _sc_gather(idx3, t32):
    mesh = plsc.VectorSubcoreMesh(core_axis_name="c", subcore_axis_name="s")

    @functools.partial(
        pl.kernel,
        mesh=mesh,
        out_type=jax.ShapeDtypeStruct((BATCH, EMBED_DIM), jnp.float32),
        scratch_types=[
            pltpu.VMEM((_IB, 128), jnp.int32),
            pltpu.VMEM((_IB, 128), jnp.int32),
            pltpu.VMEM((_BPW, EMBED_DIM), jnp.float32),
            pltpu.SemaphoreType.DMA,
        ],
        compiler_params=pltpu.CompilerParams(use_tc_tiling_on_sc=False),
    )
    def k(x_hbm, t32_hbm, out_hbm, xv, idx_v, rows_v, sem):
        wid = lax.axis_index("s") * _NUM_CORES + lax.axis_index("c")
        pltpu.sync_copy(x_hbm.at[wid], xv)
        copies = []
        for ib in range(_IB):
            for k16 in range(8):
                v = xv[ib, pl.ds(k16 * 16, 16)]
                idx_v[ib, pl.ds(k16 * 16, 16)] = ((v & (_Q - 1)) << 2) | (
                    v >> 18
                )
            copies.append(
                pltpu.async_copy(
                    t32_hbm.at[idx_v.at[ib]],
                    rows_v.at[pl.ds(ib * 128, 128)],
                    sem,
                )
            )
        for c in copies:
            c.wait()
        obase = pl.multiple_of(wid * _BPW, _BPW)
        pltpu.sync_copy(rows_v, out_hbm.at[pl.ds(obase, _BPW)])

    return k(idx3, t32)


def kernel(x, table):
    t4 = _stage_rm(table.T)
    t32 = t4.reshape(4 * _Q, EMBED_DIM)
    x3 = x.astype(jnp.int32).reshape(_NW, _IB, 128)
    return _sc_gather(x3, t32)
